# baseline (speedup 1.0000x reference)
import functools
import math

import jax
import jax.numpy as jnp
from jax import lax
from jax.experimental import pallas as pl
from jax.experimental.pallas import tpu as pltpu
from jax.experimental.pallas import tpu_sc as plsc

_HID = 128
_NFREQ = 16
_CUTOFF = 10.0
_NL = 4
_BN = 2000
_BE = 2000


def _silu(x):
    return x * jax.nn.sigmoid(x)


def _prep_body(h_ref, wa_ref, wb_ref, p_ref, q_ref):
    h = h_ref[...]
    p_ref[...] = jnp.dot(h, wa_ref[...], preferred_element_type=jnp.float32)
    q_ref[...] = jnp.dot(h, wb_ref[...], preferred_element_type=jnp.float32)


def _prep(h, wa, wb):
    n = h.shape[0]
    return pl.pallas_call(
        _prep_body,
        grid=(n // _BN,),
        in_specs=[
            pl.BlockSpec((_BN, _HID), lambda i: (i, 0)),
            pl.BlockSpec((_HID, _HID), lambda i: (0, 0)),
            pl.BlockSpec((_HID, _HID), lambda i: (0, 0)),
        ],
        out_specs=[
            pl.BlockSpec((_BN, _HID), lambda i: (i, 0)),
            pl.BlockSpec((_BN, _HID), lambda i: (i, 0)),
        ],
        out_shape=[
            jax.ShapeDtypeStruct((n, _HID), jnp.float32),
            jax.ShapeDtypeStruct((n, _HID), jnp.float32),
        ],
    )(h, wa, wb)


def _edge_body(g1_ref, g2_ref, ps_ref, pd_ref,
               w1c_ref, b1_ref, w2_ref, b2_ref,
               cw1_ref, cb1_ref, cw2t_ref,
               m_ref, msg_ref, loss_ref):
    rel = pd_ref[...] - ps_ref[...]
    d2 = jnp.sum(rel * rel, axis=1, keepdims=True) + 1e-8
    dist = jnp.sqrt(d2)
    freqs = (lax.broadcasted_iota(jnp.int32, (1, _NFREQ), 1) + 1
             ).astype(jnp.float32) * (math.pi / _CUTOFF)
    demb = jnp.sin(dist * freqs)
    pre = (g1_ref[...] + g2_ref[...]
           + jnp.dot(demb, w1c_ref[...], preferred_element_type=jnp.float32)
           + b1_ref[...])
    m1 = _silu(pre)
    m = _silu(jnp.dot(m1, w2_ref[...], preferred_element_type=jnp.float32)
              + b2_ref[...])
    w = 0.5 * (jnp.cos(dist * (math.pi / _CUTOFF)) + 1.0)
    w = w * (dist < _CUTOFF).astype(jnp.float32)
    m = m * w
    ch = _silu(jnp.dot(m, cw1_ref[...], preferred_element_type=jnp.float32)
               + cb1_ref[...])
    cs = jnp.tanh(jnp.sum(ch * cw2t_ref[...], axis=1, keepdims=True))
    m_ref[...] = m
    msg_ref[...] = rel / (dist + 1.0) * cs
    part = jnp.sum((dist - 1.5) ** 2 * w)

    @pl.when(pl.program_id(0) == 0)
    def _():
        loss_ref[...] = jnp.zeros_like(loss_ref)

    loss_ref[...] += part


def _edge(g1, g2, ps, pd, params, l):
    e = g1.shape[0]
    w1c = params[f"e{l}_W1"][2 * _HID:]
    b1 = params[f"e{l}_b1"].reshape(1, _HID)
    w2 = params[f"e{l}_W2"]
    b2 = params[f"e{l}_b2"].reshape(1, _HID)
    cw1 = params[f"c{l}_W1"]
    cb1 = params[f"c{l}_b1"].reshape(1, _HID)
    cw2t = params[f"c{l}_W2"].reshape(1, _HID)
    full = lambda shape: pl.BlockSpec(shape, lambda i: (0,) * len(shape))
    return pl.pallas_call(
        _edge_body,
        grid=(e // _BE,),
        in_specs=[
            pl.BlockSpec((_BE, _HID), lambda i: (i, 0)),
            pl.BlockSpec((_BE, _HID), lambda i: (i, 0)),
            pl.BlockSpec((_BE, 16), lambda i: (i, 0)),
            pl.BlockSpec((_BE, 16), lambda i: (i, 0)),
            full((_NFREQ, _HID)), full((1, _HID)), full((_HID, _HID)),
            full((1, _HID)), full((_HID, _HID)), full((1, _HID)),
            full((1, _HID)),
        ],
        out_specs=[
            pl.BlockSpec((_BE, _HID), lambda i: (i, 0)),
            pl.BlockSpec((_BE, 16), lambda i: (i, 0)),
            pl.BlockSpec((1, 1), lambda i: (0, 0)),
        ],
        out_shape=[
            jax.ShapeDtypeStruct((e, _HID), jnp.float32),
            jax.ShapeDtypeStruct((e, 16), jnp.float32),
            jax.ShapeDtypeStruct((1, 1), jnp.float32),
        ],
    )(g1, g2, ps, pd, w1c, b1, w2, b2, cw1, cb1, cw2t)


def _node_body(h_ref, a0_ref, a1_ref, p16_ref, d0_ref, d1_ref,
               nw1a_ref, nw1b_ref, nb1_ref, nw2_ref, nb2_ref,
               wa_ref, wb_ref,
               hn_ref, pn_ref, pp_ref, pq_ref):
    h = h_ref[...]
    agg = a0_ref[...] + a1_ref[...]
    upd = _silu(jnp.dot(h, nw1a_ref[...], preferred_element_type=jnp.float32)
                + jnp.dot(agg, nw1b_ref[...], preferred_element_type=jnp.float32)
                + nb1_ref[...])
    hn = h + jnp.dot(upd, nw2_ref[...], preferred_element_type=jnp.float32) + nb2_ref[...]
    hn_ref[...] = hn
    pn_ref[...] = p16_ref[...] + d0_ref[...] + d1_ref[...]
    pp_ref[...] = jnp.dot(hn, wa_ref[...], preferred_element_type=jnp.float32)
    pq_ref[...] = jnp.dot(hn, wb_ref[...], preferred_element_type=jnp.float32)


def _node(h, a0, a1, pos16, d0, d1, params, l, wa, wb):
    n = h.shape[0]
    nw1 = params[f"n{l}_W1"]
    nb1 = params[f"n{l}_b1"].reshape(1, _HID)
    nw2 = params[f"n{l}_W2"]
    nb2 = params[f"n{l}_b2"].reshape(1, _HID)
    full = lambda shape: pl.BlockSpec(shape, lambda i: (0,) * len(shape))
    return pl.pallas_call(
        _node_body,
        grid=(n // _BN,),
        in_specs=[
            pl.BlockSpec((_BN, _HID), lambda i: (i, 0)),
            pl.BlockSpec((_BN, _HID), lambda i: (i, 0)),
            pl.BlockSpec((_BN, _HID), lambda i: (i, 0)),
            pl.BlockSpec((_BN, 16), lambda i: (i, 0)),
            pl.BlockSpec((_BN, 16), lambda i: (i, 0)),
            pl.BlockSpec((_BN, 16), lambda i: (i, 0)),
            full((_HID, _HID)), full((_HID, _HID)), full((1, _HID)),
            full((_HID, _HID)), full((1, _HID)),
            full((_HID, _HID)), full((_HID, _HID)),
        ],
        out_specs=[
            pl.BlockSpec((_BN, _HID), lambda i: (i, 0)),
            pl.BlockSpec((_BN, 16), lambda i: (i, 0)),
            pl.BlockSpec((_BN, _HID), lambda i: (i, 0)),
            pl.BlockSpec((_BN, _HID), lambda i: (i, 0)),
        ],
        out_shape=[
            jax.ShapeDtypeStruct((n, _HID), jnp.float32),
            jax.ShapeDtypeStruct((n, 16), jnp.float32),
            jax.ShapeDtypeStruct((n, _HID), jnp.float32),
            jax.ShapeDtypeStruct((n, _HID), jnp.float32),
        ],
    )(h, a0, a1, pos16, d0, d1, nw1[:_HID], nw1[_HID:], nb1, nw2, nb2, wa, wb)


def _psum_body(p16_ref, s_ref):
    @pl.when(pl.program_id(0) == 0)
    def _():
        s_ref[...] = jnp.zeros_like(s_ref)

    s_ref[...] += jnp.sum(p16_ref[...], axis=0, keepdims=True)


def _psum(pos16):
    n = pos16.shape[0]
    return pl.pallas_call(
        _psum_body,
        grid=(n // _BN,),
        in_specs=[pl.BlockSpec((_BN, 16), lambda i: (i, 0))],
        out_specs=pl.BlockSpec((1, 16), lambda i: (0, 0)),
        out_shape=jax.ShapeDtypeStruct((1, 16), jnp.float32),
    )(pos16)


def _head_body(gp_ref, ip_ref, p16_ref, psum_ref,
               gb1_ref, gw2_ref, gb2_ref,
               iwn_ref, ib1_ref, iw2_ref, ib2_ref,
               geo_ref, inv_ref, inv_n):
    s = _silu(gp_ref[...] + gb1_ref[...])
    geo_ref[...] = jnp.dot(s, gw2_ref[...], preferred_element_type=jnp.float32) + gb2_ref[...]
    p16 = p16_ref[...]
    norm = jnp.sqrt(jnp.sum(p16 * p16, axis=1, keepdims=True))
    mean = psum_ref[...] * inv_n
    pre = (ip_ref[...] + ib1_ref[...]
           + norm * iwn_ref[0:1, :]
           + mean[0:1, 0:1] * iwn_ref[1:2, :]
           + mean[0:1, 1:2] * iwn_ref[2:3, :]
           + mean[0:1, 2:3] * iwn_ref[3:4, :])
    inv_ref[...] = jnp.dot(_silu(pre), iw2_ref[...], preferred_element_type=jnp.float32) + ib2_ref[...]


def _head(gp, ip, pos16, psum, params):
    n = gp.shape[0]
    gb1 = params["g_b1"].reshape(1, _HID)
    gw2 = params["g_W2"]
    gb2 = params["g_b2"].reshape(1, 64)
    iwn = jnp.concatenate([params["i_W1"][_HID:], jnp.zeros((4, _HID), jnp.float32)], axis=0)
    ib1 = params["i_b1"].reshape(1, _HID)
    iw2 = params["i_W2"]
    ib2 = params["i_b2"].reshape(1, _HID)
    full = lambda shape: pl.BlockSpec(shape, lambda i: (0,) * len(shape))
    return pl.pallas_call(
        functools.partial(_head_body, inv_n=1.0 / n),
        grid=(n // _BN,),
        in_specs=[
            pl.BlockSpec((_BN, _HID), lambda i: (i, 0)),
            pl.BlockSpec((_BN, _HID), lambda i: (i, 0)),
            pl.BlockSpec((_BN, 16), lambda i: (i, 0)),
            full((1, 16)),
            full((1, _HID)), full((_HID, 64)), full((1, 64)),
            full((8, _HID)), full((1, _HID)), full((_HID, _HID)), full((1, _HID)),
        ],
        out_specs=[
            pl.BlockSpec((_BN, 64), lambda i: (i, 0)),
            pl.BlockSpec((_BN, _HID), lambda i: (i, 0)),
        ],
        out_shape=[
            jax.ShapeDtypeStruct((n, 64), jnp.float32),
            jax.ShapeDtypeStruct((n, _HID), jnp.float32),
        ],
    )(gp, ip, pos16, psum, gb1, gw2, gb2, iwn, ib1, iw2, ib2)


def _gather(p, q, pos16, src, dst):
    return p[src], q[dst], pos16[src], pos16[dst]


def _scatter(m, msg, dst, n):
    agg = jax.ops.segment_sum(m, dst, num_segments=n)
    dpos = jax.ops.segment_sum(msg, dst, num_segments=n)
    z128 = jnp.zeros_like(agg)
    z16 = jnp.zeros_like(dpos)
    return agg, z128, dpos, z16


def kernel(h, pos, batch, edge_index, params):
    del batch
    n = h.shape[0]
    e = edge_index.shape[1]
    src = edge_index[0].astype(jnp.int32)
    dst = edge_index[1].astype(jnp.int32)
    pos16 = jnp.concatenate([pos.astype(jnp.float32),
                             jnp.zeros((n, 13), jnp.float32)], axis=1)
    h = h.astype(jnp.float32)

    p, q = _prep(h, params["e0_W1"][:_HID], params["e0_W1"][_HID:2 * _HID])
    losses = []
    for l in range(_NL):
        g1, g2, ps, pd = _gather(p, q, pos16, src, dst)
        m, msg, lpart = _edge(g1, g2, ps, pd, params, l)
        a0, a1, d0, d1 = _scatter(m, msg, dst, n)
        if l < _NL - 1:
            wa = params[f"e{l + 1}_W1"][:_HID]
            wb = params[f"e{l + 1}_W1"][_HID:2 * _HID]
        else:
            wa = params["g_W1"]
            wb = params["i_W1"][:_HID]
        h, pos16, p, q = _node(h, a0, a1, pos16, d0, d1, params, l, wa, wb)
        losses.append(lpart)

    psum = _psum(pos16)
    geo, inv = _head(p, q, pos16, psum, params)
    closs = ((losses[0] + losses[1] + losses[2] + losses[3])[0, 0] / e).astype(jnp.float32)
    return h, pos16[:, :3], geo, inv, closs


# speedup vs baseline: 1.6709x; 1.6709x over previous
import functools
import math

import jax
import jax.numpy as jnp
from jax import lax
from jax.experimental import pallas as pl
from jax.experimental.pallas import tpu as pltpu
from jax.experimental.pallas import tpu_sc as plsc

_HID = 128
_NFREQ = 16
_CUTOFF = 10.0
_NL = 4
_BN = 2000
_BE = 2000
_D = _HID + 16

_NC = 2
_NS = 16
_NW = _NC * _NS
_CHUNK = 80


def _silu(x):
    return x * jax.nn.sigmoid(x)


def _prep_body(h_ref, p16_ref, wa_ref, wb_ref, t1_ref, t2_ref):
    h = h_ref[...]
    p16 = p16_ref[...]
    t1_ref[...] = jnp.concatenate(
        [jnp.dot(h, wa_ref[...], preferred_element_type=jnp.float32), p16], axis=1)
    t2_ref[...] = jnp.concatenate(
        [jnp.dot(h, wb_ref[...], preferred_element_type=jnp.float32), p16], axis=1)


def _prep(h, pos16, wa, wb):
    n = h.shape[0]
    return pl.pallas_call(
        _prep_body,
        grid=(n // _BN,),
        in_specs=[
            pl.BlockSpec((_BN, _HID), lambda i: (i, 0)),
            pl.BlockSpec((_BN, 16), lambda i: (i, 0)),
            pl.BlockSpec((_HID, _HID), lambda i: (0, 0)),
            pl.BlockSpec((_HID, _HID), lambda i: (0, 0)),
        ],
        out_specs=[
            pl.BlockSpec((_BN, _D), lambda i: (i, 0)),
            pl.BlockSpec((_BN, _D), lambda i: (i, 0)),
        ],
        out_shape=[
            jax.ShapeDtypeStruct((n, _D), jnp.float32),
            jax.ShapeDtypeStruct((n, _D), jnp.float32),
        ],
    )(h, pos16, wa, wb)


def _sc_gather(t1, t2, src, dst):
    n, d = t1.shape
    e = src.shape[0]
    per_w = e // _NW
    nchunks = per_w // _CHUNK
    mesh = plsc.VectorSubcoreMesh(core_axis_name="c", subcore_axis_name="s")

    @functools.partial(
        pl.kernel,
        mesh=mesh,
        compiler_params=pltpu.CompilerParams(use_tc_tiling_on_sc=False),
        out_type=[jax.ShapeDtypeStruct((e, d), jnp.float32),
                  jax.ShapeDtypeStruct((e, d), jnp.float32)],
        scratch_types=[pltpu.VMEM((_CHUNK,), jnp.int32),
                       pltpu.VMEM((_CHUNK,), jnp.int32),
                       pltpu.VMEM((_CHUNK, d), jnp.float32),
                       pltpu.VMEM((_CHUNK, d), jnp.float32),
                       pltpu.SemaphoreType.DMA],
    )
    def k(t1_hbm, t2_hbm, src_hbm, dst_hbm, g1_hbm, g2_hbm, si, di, r1, r2, sem):
        wid = lax.axis_index("s") * _NC + lax.axis_index("c")
        base = wid * per_w

        def body(j, carry):
            off = base + j * _CHUNK
            pltpu.sync_copy(src_hbm.at[pl.ds(off, _CHUNK)], si)
            pltpu.sync_copy(dst_hbm.at[pl.ds(off, _CHUNK)], di)
            c1 = pltpu.async_copy(t1_hbm.at[si], r1, sem)
            c2 = pltpu.async_copy(t2_hbm.at[di], r2, sem)
            c1.wait()
            c2.wait()
            pltpu.sync_copy(r1, g1_hbm.at[pl.ds(off, _CHUNK)])
            pltpu.sync_copy(r2, g2_hbm.at[pl.ds(off, _CHUNK)])
            return carry

        lax.fori_loop(0, nchunks, body, 0)

    return k(t1, t2, src, dst)


def _sc_scatter(mm, dst, zeros):
    e, d = mm.shape
    n = zeros.shape[0]
    per_w = e // _NW
    nchunks = per_w // _CHUNK
    rows_t = n // _NS
    mesh = plsc.VectorSubcoreMesh(core_axis_name="c", subcore_axis_name="s")

    @functools.partial(
        pl.kernel,
        mesh=mesh,
        compiler_params=pltpu.CompilerParams(use_tc_tiling_on_sc=False),
        out_type=jax.ShapeDtypeStruct((_NC, n, d), jnp.float32),
        scratch_types=[pltpu.VMEM((_CHUNK,), jnp.int32),
                       pltpu.VMEM((_CHUNK, d), jnp.float32),
                       pltpu.VMEM_SHARED((n, d), jnp.float32),
                       pltpu.SemaphoreType.DMA],
    )
    def k(mm_hbm, dst_hbm, z_hbm, out_hbm, di, rv, acc, sem):
        c = lax.axis_index("c")
        s = lax.axis_index("s")
        wid = s * _NC + c
        pltpu.sync_copy(z_hbm.at[pl.ds(s * rows_t, rows_t)],
                        acc.at[pl.ds(s * rows_t, rows_t)])
        plsc.subcore_barrier()
        base = wid * per_w

        def body(j, carry):
            off = base + j * _CHUNK
            pltpu.sync_copy(dst_hbm.at[pl.ds(off, _CHUNK)], di)
            pltpu.sync_copy(mm_hbm.at[pl.ds(off, _CHUNK)], rv)
            pltpu.sync_copy(rv, acc.at[di], add=True)
            return carry

        lax.fori_loop(0, nchunks, body, 0)
        plsc.subcore_barrier()
        pltpu.sync_copy(acc.at[pl.ds(s * rows_t, rows_t)],
                        out_hbm.at[c, pl.ds(s * rows_t, rows_t)])

    return k(mm, dst, zeros)


def _edge_body(t1g_ref, t2g_ref,
               w1c_ref, b1_ref, w2_ref, b2_ref,
               cw1_ref, cb1_ref, cw2t_ref,
               mm_ref, loss_ref):
    t1g = t1g_ref[...]
    t2g = t2g_ref[...]
    ps = t1g[:, _HID:]
    pd = t2g[:, _HID:]
    rel = pd - ps
    d2 = jnp.sum(rel * rel, axis=1, keepdims=True) + 1e-8
    dist = jnp.sqrt(d2)
    freqs = (lax.broadcasted_iota(jnp.int32, (1, _NFREQ), 1) + 1
             ).astype(jnp.float32) * (math.pi / _CUTOFF)
    demb = jnp.sin(dist * freqs)
    pre = (t1g[:, :_HID] + t2g[:, :_HID]
           + jnp.dot(demb, w1c_ref[...], preferred_element_type=jnp.float32)
           + b1_ref[...])
    m1 = _silu(pre)
    m = _silu(jnp.dot(m1, w2_ref[...], preferred_element_type=jnp.float32)
              + b2_ref[...])
    w = 0.5 * (jnp.cos(dist * (math.pi / _CUTOFF)) + 1.0)
    w = w * (dist < _CUTOFF).astype(jnp.float32)
    m = m * w
    ch = _silu(jnp.dot(m, cw1_ref[...], preferred_element_type=jnp.float32)
               + cb1_ref[...])
    cs = jnp.tanh(jnp.sum(ch * cw2t_ref[...], axis=1, keepdims=True))
    msg = rel / (dist + 1.0) * cs
    mm_ref[...] = jnp.concatenate([m, msg], axis=1)
    part = jnp.sum((dist - 1.5) ** 2 * w)

    @pl.when(pl.program_id(0) == 0)
    def _():
        loss_ref[...] = jnp.zeros_like(loss_ref)

    loss_ref[...] += part


def _edge(g1, g2, params, l):
    e = g1.shape[0]
    w1c = params[f"e{l}_W1"][2 * _HID:]
    b1 = params[f"e{l}_b1"].reshape(1, _HID)
    w2 = params[f"e{l}_W2"]
    b2 = params[f"e{l}_b2"].reshape(1, _HID)
    cw1 = params[f"c{l}_W1"]
    cb1 = params[f"c{l}_b1"].reshape(1, _HID)
    cw2t = params[f"c{l}_W2"].reshape(1, _HID)
    full = lambda shape: pl.BlockSpec(shape, lambda i: (0,) * len(shape))
    return pl.pallas_call(
        _edge_body,
        grid=(e // _BE,),
        in_specs=[
            pl.BlockSpec((_BE, _D), lambda i: (i, 0)),
            pl.BlockSpec((_BE, _D), lambda i: (i, 0)),
            full((_NFREQ, _HID)), full((1, _HID)), full((_HID, _HID)),
            full((1, _HID)), full((_HID, _HID)), full((1, _HID)),
            full((1, _HID)),
        ],
        out_specs=[
            pl.BlockSpec((_BE, _D), lambda i: (i, 0)),
            pl.BlockSpec((1, 1), lambda i: (0, 0)),
        ],
        out_shape=[
            jax.ShapeDtypeStruct((e, _D), jnp.float32),
            jax.ShapeDtypeStruct((1, 1), jnp.float32),
        ],
    )(g1, g2, w1c, b1, w2, b2, cw1, cb1, cw2t)


def _node_body(h_ref, a0_ref, a1_ref, t1in_ref,
               nw1a_ref, nw1b_ref, nb1_ref, nw2_ref, nb2_ref,
               wa_ref, wb_ref,
               hn_ref, t1_ref, t2_ref):
    h = h_ref[...]
    acc = a0_ref[0] + a1_ref[0]
    agg = acc[:, :_HID]
    upd = _silu(jnp.dot(h, nw1a_ref[...], preferred_element_type=jnp.float32)
                + jnp.dot(agg, nw1b_ref[...], preferred_element_type=jnp.float32)
                + nb1_ref[...])
    hn = h + jnp.dot(upd, nw2_ref[...], preferred_element_type=jnp.float32) + nb2_ref[...]
    pn = t1in_ref[:, _HID:] + acc[:, _HID:]
    hn_ref[...] = hn
    t1_ref[...] = jnp.concatenate(
        [jnp.dot(hn, wa_ref[...], preferred_element_type=jnp.float32), pn], axis=1)
    t2_ref[...] = jnp.concatenate(
        [jnp.dot(hn, wb_ref[...], preferred_element_type=jnp.float32), pn], axis=1)


def _node(h, agg2, t1, params, l, wa, wb):
    n = h.shape[0]
    nw1 = params[f"n{l}_W1"]
    nb1 = params[f"n{l}_b1"].reshape(1, _HID)
    nw2 = params[f"n{l}_W2"]
    nb2 = params[f"n{l}_b2"].reshape(1, _HID)
    full = lambda shape: pl.BlockSpec(shape, lambda i: (0,) * len(shape))
    return pl.pallas_call(
        _node_body,
        grid=(n // _BN,),
        in_specs=[
            pl.BlockSpec((_BN, _HID), lambda i: (i, 0)),
            pl.BlockSpec((1, _BN, _D), lambda i: (0, i, 0)),
            pl.BlockSpec((1, _BN, _D), lambda i: (1, i, 0)),
            pl.BlockSpec((_BN, _D), lambda i: (i, 0)),
            full((_HID, _HID)), full((_HID, _HID)), full((1, _HID)),
            full((_HID, _HID)), full((1, _HID)),
            full((_HID, _HID)), full((_HID, _HID)),
        ],
        out_specs=[
            pl.BlockSpec((_BN, _HID), lambda i: (i, 0)),
            pl.BlockSpec((_BN, _D), lambda i: (i, 0)),
            pl.BlockSpec((_BN, _D), lambda i: (i, 0)),
        ],
        out_shape=[
            jax.ShapeDtypeStruct((n, _HID), jnp.float32),
            jax.ShapeDtypeStruct((n, _D), jnp.float32),
            jax.ShapeDtypeStruct((n, _D), jnp.float32),
        ],
    )(h, agg2, agg2, t1, nw1[:_HID], nw1[_HID:], nb1, nw2, nb2, wa, wb)


def _psum_body(t1_ref, s_ref):
    @pl.when(pl.program_id(0) == 0)
    def _():
        s_ref[...] = jnp.zeros_like(s_ref)

    s_ref[...] += jnp.sum(t1_ref[:, _HID:], axis=0, keepdims=True)


def _psum(t1):
    n = t1.shape[0]
    return pl.pallas_call(
        _psum_body,
        grid=(n // _BN,),
        in_specs=[pl.BlockSpec((_BN, _D), lambda i: (i, 0))],
        out_specs=pl.BlockSpec((1, 16), lambda i: (0, 0)),
        out_shape=jax.ShapeDtypeStruct((1, 16), jnp.float32),
    )(t1)


def _head_body(t1_ref, t2_ref, psum_ref,
               gb1_ref, gw2_ref, gb2_ref,
               iwn_ref, ib1_ref, iw2_ref, ib2_ref,
               geo_ref, inv_ref, inv_n):
    t1 = t1_ref[...]
    t2 = t2_ref[...]
    s = _silu(t1[:, :_HID] + gb1_ref[...])
    geo_ref[...] = jnp.dot(s, gw2_ref[...], preferred_element_type=jnp.float32) + gb2_ref[...]
    p16 = t1[:, _HID:]
    norm = jnp.sqrt(jnp.sum(p16 * p16, axis=1, keepdims=True))
    mean = psum_ref[...] * inv_n
    pre = (t2[:, :_HID] + ib1_ref[...]
           + norm * iwn_ref[0:1, :]
           + mean[0:1, 0:1] * iwn_ref[1:2, :]
           + mean[0:1, 1:2] * iwn_ref[2:3, :]
           + mean[0:1, 2:3] * iwn_ref[3:4, :])
    inv_ref[...] = jnp.dot(_silu(pre), iw2_ref[...], preferred_element_type=jnp.float32) + ib2_ref[...]


def _head(t1, t2, psum, params):
    n = t1.shape[0]
    gb1 = params["g_b1"].reshape(1, _HID)
    gw2 = params["g_W2"]
    gb2 = params["g_b2"].reshape(1, 64)
    iwn = jnp.concatenate([params["i_W1"][_HID:], jnp.zeros((4, _HID), jnp.float32)], axis=0)
    ib1 = params["i_b1"].reshape(1, _HID)
    iw2 = params["i_W2"]
    ib2 = params["i_b2"].reshape(1, _HID)
    full = lambda shape: pl.BlockSpec(shape, lambda i: (0,) * len(shape))
    return pl.pallas_call(
        functools.partial(_head_body, inv_n=1.0 / n),
        grid=(n // _BN,),
        in_specs=[
            pl.BlockSpec((_BN, _D), lambda i: (i, 0)),
            pl.BlockSpec((_BN, _D), lambda i: (i, 0)),
            full((1, 16)),
            full((1, _HID)), full((_HID, 64)), full((1, 64)),
            full((8, _HID)), full((1, _HID)), full((_HID, _HID)), full((1, _HID)),
        ],
        out_specs=[
            pl.BlockSpec((_BN, 64), lambda i: (i, 0)),
            pl.BlockSpec((_BN, _HID), lambda i: (i, 0)),
        ],
        out_shape=[
            jax.ShapeDtypeStruct((n, 64), jnp.float32),
            jax.ShapeDtypeStruct((n, _HID), jnp.float32),
        ],
    )(t1, t2, psum, gb1, gw2, gb2, iwn, ib1, iw2, ib2)


def kernel(h, pos, batch, edge_index, params):
    del batch
    n = h.shape[0]
    e = edge_index.shape[1]
    src = edge_index[0].astype(jnp.int32)
    dst = edge_index[1].astype(jnp.int32)
    pos16 = jnp.concatenate([pos.astype(jnp.float32),
                             jnp.zeros((n, 13), jnp.float32)], axis=1)
    h = h.astype(jnp.float32)
    zeros144 = jnp.zeros((n, _D), jnp.float32)

    t1, t2 = _prep(h, pos16, params["e0_W1"][:_HID], params["e0_W1"][_HID:2 * _HID])
    losses = []
    for l in range(_NL):
        g1, g2 = _sc_gather(t1, t2, src, dst)
        mm, lpart = _edge(g1, g2, params, l)
        agg2 = _sc_scatter(mm, dst, zeros144)
        if l < _NL - 1:
            wa = params[f"e{l + 1}_W1"][:_HID]
            wb = params[f"e{l + 1}_W1"][_HID:2 * _HID]
        else:
            wa = params["g_W1"]
            wb = params["i_W1"][:_HID]
        h, t1, t2 = _node(h, agg2, t1, params, l, wa, wb)
        losses.append(lpart)

    psum = _psum(t1)
    geo, inv = _head(t1, t2, psum, params)
    closs = ((losses[0] + losses[1] + losses[2] + losses[3])[0, 0] / e).astype(jnp.float32)
    return h, t1[:, _HID:_HID + 3], geo, inv, closs


# speedup vs baseline: 2.6532x; 1.5879x over previous
import functools
import math

import jax
import jax.numpy as jnp
from jax import lax
from jax.experimental import pallas as pl
from jax.experimental.pallas import tpu as pltpu
from jax.experimental.pallas import tpu_sc as plsc

_HID = 128
_NFREQ = 16
_CUTOFF = 10.0
_NL = 4
_BN = 2000
_BE = 2000
_D = _HID + 16

_NC = 2
_NS = 16
_NW = _NC * _NS
_CHUNK = 80
_UNROLL = 5


_TWO_PI = 6.283185307179586
_S1 = 6.2831853071795864769
_S3 = -41.341702240399755
_S5 = 81.60524927607504
_S7 = -76.70585975306136
_S9 = 42.05869394489765


def _fast_sin(x):
    y = x * (1.0 / _TWO_PI)
    k = (y + 12582912.0) - 12582912.0
    f = y - k
    a = jnp.abs(f)
    g = jnp.minimum(a, 0.5 - a)
    u = g * g
    p = g * (_S1 + u * (_S3 + u * (_S5 + u * (_S7 + u * _S9))))
    return jnp.where(f < 0.0, -p, p)


def _silu(x):
    return x * (0.5 * jnp.tanh(0.5 * x) + 0.5)


def _prep_body(h_ref, p16_ref, wa_ref, wb_ref, t1_ref, t2_ref):
    h = h_ref[...]
    p16 = p16_ref[...]
    t1_ref[...] = jnp.concatenate(
        [jnp.dot(h, wa_ref[...], preferred_element_type=jnp.float32), p16], axis=1)
    t2_ref[...] = jnp.concatenate(
        [jnp.dot(h, wb_ref[...], preferred_element_type=jnp.float32), p16], axis=1)


def _prep(h, pos16, wa, wb):
    n = h.shape[0]
    return pl.pallas_call(
        _prep_body,
        grid=(n // _BN,),
        in_specs=[
            pl.BlockSpec((_BN, _HID), lambda i: (i, 0)),
            pl.BlockSpec((_BN, 16), lambda i: (i, 0)),
            pl.BlockSpec((_HID, _HID), lambda i: (0, 0)),
            pl.BlockSpec((_HID, _HID), lambda i: (0, 0)),
        ],
        out_specs=[
            pl.BlockSpec((_BN, _D), lambda i: (i, 0)),
            pl.BlockSpec((_BN, _D), lambda i: (i, 0)),
        ],
        out_shape=[
            jax.ShapeDtypeStruct((n, _D), jnp.float32),
            jax.ShapeDtypeStruct((n, _D), jnp.float32),
        ],
    )(h, pos16, wa, wb)


def _sc_gather(t1, t2, src, dst):
    n, d = t1.shape
    e = src.shape[0]
    per_w = e // _NW
    group = _UNROLL * _CHUNK
    ngroups = per_w // group
    mesh = plsc.VectorSubcoreMesh(core_axis_name="c", subcore_axis_name="s")

    scratch = ([pltpu.VMEM((group,), jnp.int32), pltpu.VMEM((group,), jnp.int32)]
               + [pltpu.VMEM((_CHUNK, d), jnp.float32) for _ in range(2 * _UNROLL)]
               + [pltpu.SemaphoreType.DMA, pltpu.SemaphoreType.DMA,
                  pltpu.SemaphoreType.DMA])

    @functools.partial(
        pl.kernel,
        mesh=mesh,
        compiler_params=pltpu.CompilerParams(use_tc_tiling_on_sc=False),
        out_type=[jax.ShapeDtypeStruct((e, d), jnp.float32),
                  jax.ShapeDtypeStruct((e, d), jnp.float32)],
        scratch_types=scratch,
    )
    def k(t1_hbm, t2_hbm, src_hbm, dst_hbm, g1_hbm, g2_hbm, *sc):
        si, di = sc[0], sc[1]
        r1 = sc[2:2 + _UNROLL]
        r2 = sc[2 + _UNROLL:2 + 2 * _UNROLL]
        isem, gsem, wsem = sc[-3], sc[-2], sc[-1]
        wid = lax.axis_index("s") * _NC + lax.axis_index("c")
        base = wid * per_w

        def body(t, carry):
            off0 = base + t * group
            ci = pltpu.async_copy(src_hbm.at[pl.ds(off0, group)], si, isem)
            cj = pltpu.async_copy(dst_hbm.at[pl.ds(off0, group)], di, isem)
            ci.wait()
            cj.wait()
            gs = []
            for u in range(_UNROLL):
                gs.append(pltpu.async_copy(
                    t1_hbm.at[si.at[pl.ds(u * _CHUNK, _CHUNK)]], r1[u], gsem))
                gs.append(pltpu.async_copy(
                    t2_hbm.at[di.at[pl.ds(u * _CHUNK, _CHUNK)]], r2[u], gsem))
            ws = []
            for u in range(_UNROLL):
                gs[2 * u].wait()
                gs[2 * u + 1].wait()
                off = off0 + u * _CHUNK
                ws.append(pltpu.async_copy(r1[u], g1_hbm.at[pl.ds(off, _CHUNK)], wsem))
                ws.append(pltpu.async_copy(r2[u], g2_hbm.at[pl.ds(off, _CHUNK)], wsem))
            for cpy in ws:
                cpy.wait()
            return carry

        lax.fori_loop(0, ngroups, body, 0)

    return k(t1, t2, src, dst)


def _sc_scatter(mm, dst, zeros):
    e, d = mm.shape
    n = zeros.shape[0]
    per_w = e // _NW
    chunk = 40
    group = _UNROLL * chunk
    ngroups = per_w // group
    rows_t = n // _NS
    mesh = plsc.VectorSubcoreMesh(core_axis_name="c", subcore_axis_name="s")

    scratch = ([pltpu.VMEM((chunk,), jnp.int32) for _ in range(_UNROLL)]
               + [pltpu.VMEM((chunk, d), jnp.float32) for _ in range(_UNROLL)]
               + [pltpu.VMEM_SHARED((n, d), jnp.float32)]
               + [pltpu.SemaphoreType.DMA, pltpu.SemaphoreType.DMA,
                  pltpu.SemaphoreType.DMA])

    @functools.partial(
        pl.kernel,
        mesh=mesh,
        compiler_params=pltpu.CompilerParams(use_tc_tiling_on_sc=False),
        out_type=jax.ShapeDtypeStruct((_NC, n, d), jnp.float32),
        scratch_types=scratch,
    )
    def k(mm_hbm, dst_hbm, z_hbm, out_hbm, *sc):
        di = sc[0:_UNROLL]
        rv = sc[_UNROLL:2 * _UNROLL]
        acc = sc[2 * _UNROLL]
        isem, lsem, ssem = sc[-3], sc[-2], sc[-1]
        c = lax.axis_index("c")
        s = lax.axis_index("s")
        wid = s * _NC + c
        pltpu.sync_copy(z_hbm.at[pl.ds(s * rows_t, rows_t)],
                        acc.at[pl.ds(s * rows_t, rows_t)])
        plsc.subcore_barrier()
        base = wid * per_w

        def body(t, carry):
            off0 = base + t * group
            ics, lcs = [], []
            for u in range(_UNROLL):
                off = off0 + u * chunk
                ics.append(pltpu.async_copy(dst_hbm.at[pl.ds(off, chunk)], di[u], isem))
                lcs.append(pltpu.async_copy(mm_hbm.at[pl.ds(off, chunk)], rv[u], lsem))
            scs = []
            for u in range(_UNROLL):
                ics[u].wait()
                lcs[u].wait()
                scs.append(pltpu.async_copy(rv[u], acc.at[di[u]], ssem, add=True))
            for cpy in scs:
                cpy.wait()
            return carry

        lax.fori_loop(0, ngroups, body, 0)
        plsc.subcore_barrier()
        pltpu.sync_copy(acc.at[pl.ds(s * rows_t, rows_t)],
                        out_hbm.at[c, pl.ds(s * rows_t, rows_t)])

    return k(mm, dst, zeros)


def _edge_body(t1g_ref, t2g_ref,
               w1c_ref, b1_ref, w2_ref, b2_ref,
               cw1_ref, cb1_ref, cw2t_ref,
               mm_ref, loss_ref):
    t1g = t1g_ref[...]
    t2g = t2g_ref[...]
    ps = t1g[:, _HID:]
    pd = t2g[:, _HID:]
    rel = pd - ps
    d2 = jnp.sum(rel * rel, axis=1, keepdims=True) + 1e-8
    dist = jnp.sqrt(d2)
    freqs = (lax.broadcasted_iota(jnp.int32, (1, _NFREQ), 1) + 1
             ).astype(jnp.float32) * (math.pi / _CUTOFF)
    demb = _fast_sin(dist * freqs)
    pre = (t1g[:, :_HID] + t2g[:, :_HID]
           + jnp.dot(demb, w1c_ref[...], preferred_element_type=jnp.float32)
           + b1_ref[...])
    m1 = _silu(pre)
    m = _silu(jnp.dot(m1, w2_ref[...], preferred_element_type=jnp.float32)
              + b2_ref[...])
    w = 0.5 * (_fast_sin(dist * (math.pi / _CUTOFF) + (0.5 * math.pi)) + 1.0)
    w = w * (dist < _CUTOFF).astype(jnp.float32)
    m = m * w
    ch = _silu(jnp.dot(m, cw1_ref[...], preferred_element_type=jnp.float32)
               + cb1_ref[...])
    cs = jnp.tanh(jnp.sum(ch * cw2t_ref[...], axis=1, keepdims=True))
    msg = rel / (dist + 1.0) * cs
    mm_ref[...] = jnp.concatenate([m, msg], axis=1)
    part = jnp.sum((dist - 1.5) ** 2 * w)

    @pl.when(pl.program_id(0) == 0)
    def _():
        loss_ref[...] = jnp.zeros_like(loss_ref)

    loss_ref[...] += part


def _edge(g1, g2, params, l):
    e = g1.shape[0]
    w1c = params[f"e{l}_W1"][2 * _HID:]
    b1 = params[f"e{l}_b1"].reshape(1, _HID)
    w2 = params[f"e{l}_W2"]
    b2 = params[f"e{l}_b2"].reshape(1, _HID)
    cw1 = params[f"c{l}_W1"]
    cb1 = params[f"c{l}_b1"].reshape(1, _HID)
    cw2t = params[f"c{l}_W2"].reshape(1, _HID)
    full = lambda shape: pl.BlockSpec(shape, lambda i: (0,) * len(shape))
    return pl.pallas_call(
        _edge_body,
        grid=(e // _BE,),
        in_specs=[
            pl.BlockSpec((_BE, _D), lambda i: (i, 0)),
            pl.BlockSpec((_BE, _D), lambda i: (i, 0)),
            full((_NFREQ, _HID)), full((1, _HID)), full((_HID, _HID)),
            full((1, _HID)), full((_HID, _HID)), full((1, _HID)),
            full((1, _HID)),
        ],
        out_specs=[
            pl.BlockSpec((_BE, _D), lambda i: (i, 0)),
            pl.BlockSpec((1, 1), lambda i: (0, 0)),
        ],
        out_shape=[
            jax.ShapeDtypeStruct((e, _D), jnp.float32),
            jax.ShapeDtypeStruct((1, 1), jnp.float32),
        ],
    )(g1, g2, w1c, b1, w2, b2, cw1, cb1, cw2t)


def _node_body(h_ref, a0_ref, a1_ref, t1in_ref,
               nw1a_ref, nw1b_ref, nb1_ref, nw2_ref, nb2_ref,
               wa_ref, wb_ref,
               hn_ref, t1_ref, t2_ref):
    h = h_ref[...]
    acc = a0_ref[0] + a1_ref[0]
    agg = acc[:, :_HID]
    upd = _silu(jnp.dot(h, nw1a_ref[...], preferred_element_type=jnp.float32)
                + jnp.dot(agg, nw1b_ref[...], preferred_element_type=jnp.float32)
                + nb1_ref[...])
    hn = h + jnp.dot(upd, nw2_ref[...], preferred_element_type=jnp.float32) + nb2_ref[...]
    pn = t1in_ref[:, _HID:] + acc[:, _HID:]
    hn_ref[...] = hn
    t1_ref[...] = jnp.concatenate(
        [jnp.dot(hn, wa_ref[...], preferred_element_type=jnp.float32), pn], axis=1)
    t2_ref[...] = jnp.concatenate(
        [jnp.dot(hn, wb_ref[...], preferred_element_type=jnp.float32), pn], axis=1)


def _node(h, agg2, t1, params, l, wa, wb):
    n = h.shape[0]
    nw1 = params[f"n{l}_W1"]
    nb1 = params[f"n{l}_b1"].reshape(1, _HID)
    nw2 = params[f"n{l}_W2"]
    nb2 = params[f"n{l}_b2"].reshape(1, _HID)
    full = lambda shape: pl.BlockSpec(shape, lambda i: (0,) * len(shape))
    return pl.pallas_call(
        _node_body,
        grid=(n // _BN,),
        in_specs=[
            pl.BlockSpec((_BN, _HID), lambda i: (i, 0)),
            pl.BlockSpec((1, _BN, _D), lambda i: (0, i, 0)),
            pl.BlockSpec((1, _BN, _D), lambda i: (1, i, 0)),
            pl.BlockSpec((_BN, _D), lambda i: (i, 0)),
            full((_HID, _HID)), full((_HID, _HID)), full((1, _HID)),
            full((_HID, _HID)), full((1, _HID)),
            full((_HID, _HID)), full((_HID, _HID)),
        ],
        out_specs=[
            pl.BlockSpec((_BN, _HID), lambda i: (i, 0)),
            pl.BlockSpec((_BN, _D), lambda i: (i, 0)),
            pl.BlockSpec((_BN, _D), lambda i: (i, 0)),
        ],
        out_shape=[
            jax.ShapeDtypeStruct((n, _HID), jnp.float32),
            jax.ShapeDtypeStruct((n, _D), jnp.float32),
            jax.ShapeDtypeStruct((n, _D), jnp.float32),
        ],
    )(h, agg2, agg2, t1, nw1[:_HID], nw1[_HID:], nb1, nw2, nb2, wa, wb)


def _psum_body(t1_ref, s_ref):
    @pl.when(pl.program_id(0) == 0)
    def _():
        s_ref[...] = jnp.zeros_like(s_ref)

    s_ref[...] += jnp.sum(t1_ref[:, _HID:], axis=0, keepdims=True)


def _psum(t1):
    n = t1.shape[0]
    return pl.pallas_call(
        _psum_body,
        grid=(n // _BN,),
        in_specs=[pl.BlockSpec((_BN, _D), lambda i: (i, 0))],
        out_specs=pl.BlockSpec((1, 16), lambda i: (0, 0)),
        out_shape=jax.ShapeDtypeStruct((1, 16), jnp.float32),
    )(t1)


def _head_body(t1_ref, t2_ref, psum_ref,
               gb1_ref, gw2_ref, gb2_ref,
               iwn_ref, ib1_ref, iw2_ref, ib2_ref,
               geo_ref, inv_ref, inv_n):
    t1 = t1_ref[...]
    t2 = t2_ref[...]
    s = _silu(t1[:, :_HID] + gb1_ref[...])
    geo_ref[...] = jnp.dot(s, gw2_ref[...], preferred_element_type=jnp.float32) + gb2_ref[...]
    p16 = t1[:, _HID:]
    norm = jnp.sqrt(jnp.sum(p16 * p16, axis=1, keepdims=True))
    mean = psum_ref[...] * inv_n
    pre = (t2[:, :_HID] + ib1_ref[...]
           + norm * iwn_ref[0:1, :]
           + mean[0:1, 0:1] * iwn_ref[1:2, :]
           + mean[0:1, 1:2] * iwn_ref[2:3, :]
           + mean[0:1, 2:3] * iwn_ref[3:4, :])
    inv_ref[...] = jnp.dot(_silu(pre), iw2_ref[...], preferred_element_type=jnp.float32) + ib2_ref[...]


def _head(t1, t2, psum, params):
    n = t1.shape[0]
    gb1 = params["g_b1"].reshape(1, _HID)
    gw2 = params["g_W2"]
    gb2 = params["g_b2"].reshape(1, 64)
    iwn = jnp.concatenate([params["i_W1"][_HID:], jnp.zeros((4, _HID), jnp.float32)], axis=0)
    ib1 = params["i_b1"].reshape(1, _HID)
    iw2 = params["i_W2"]
    ib2 = params["i_b2"].reshape(1, _HID)
    full = lambda shape: pl.BlockSpec(shape, lambda i: (0,) * len(shape))
    return pl.pallas_call(
        functools.partial(_head_body, inv_n=1.0 / n),
        grid=(n // _BN,),
        in_specs=[
            pl.BlockSpec((_BN, _D), lambda i: (i, 0)),
            pl.BlockSpec((_BN, _D), lambda i: (i, 0)),
            full((1, 16)),
            full((1, _HID)), full((_HID, 64)), full((1, 64)),
            full((8, _HID)), full((1, _HID)), full((_HID, _HID)), full((1, _HID)),
        ],
        out_specs=[
            pl.BlockSpec((_BN, 64), lambda i: (i, 0)),
            pl.BlockSpec((_BN, _HID), lambda i: (i, 0)),
        ],
        out_shape=[
            jax.ShapeDtypeStruct((n, 64), jnp.float32),
            jax.ShapeDtypeStruct((n, _HID), jnp.float32),
        ],
    )(t1, t2, psum, gb1, gw2, gb2, iwn, ib1, iw2, ib2)


def kernel(h, pos, batch, edge_index, params):
    del batch
    n = h.shape[0]
    e = edge_index.shape[1]
    src = edge_index[0].astype(jnp.int32)
    dst = edge_index[1].astype(jnp.int32)
    pos16 = jnp.concatenate([pos.astype(jnp.float32),
                             jnp.zeros((n, 13), jnp.float32)], axis=1)
    h = h.astype(jnp.float32)
    zeros144 = jnp.zeros((n, _D), jnp.float32)

    t1, t2 = _prep(h, pos16, params["e0_W1"][:_HID], params["e0_W1"][_HID:2 * _HID])
    losses = []
    for l in range(_NL):
        g1, g2 = _sc_gather(t1, t2, src, dst)
        mm, lpart = _edge(g1, g2, params, l)
        agg2 = _sc_scatter(mm, dst, zeros144)
        if l < _NL - 1:
            wa = params[f"e{l + 1}_W1"][:_HID]
            wb = params[f"e{l + 1}_W1"][_HID:2 * _HID]
        else:
            wa = params["g_W1"]
            wb = params["i_W1"][:_HID]
        h, t1, t2 = _node(h, agg2, t1, params, l, wa, wb)
        losses.append(lpart)

    psum = _psum(t1)
    geo, inv = _head(t1, t2, psum, params)
    closs = ((losses[0] + losses[1] + losses[2] + losses[3])[0, 0] / e).astype(jnp.float32)
    return h, t1[:, _HID:_HID + 3], geo, inv, closs


# speedup vs baseline: 4.1257x; 1.5550x over previous
import functools
import math

import jax
import jax.numpy as jnp
from jax import lax
from jax.experimental import pallas as pl
from jax.experimental.pallas import tpu as pltpu
from jax.experimental.pallas import tpu_sc as plsc

_HID = 128
_NFREQ = 16
_CUTOFF = 10.0
_NL = 4
_BN = 2000
_BE = 2000

_NC = 2
_NS = 16
_NW = _NC * _NS
_CHUNK = 80
_UNROLL = 5

_TWO_PI = 6.283185307179586
_S1 = 6.2831853071795864769
_S3 = -41.341702240399755
_S5 = 81.60524927607504
_S7 = -76.70585975306136
_S9 = 42.05869394489765


def _fast_sin(x):
    y = x * (1.0 / _TWO_PI)
    k = (y + 12582912.0) - 12582912.0
    f = y - k
    a = jnp.abs(f)
    g = jnp.minimum(a, 0.5 - a)
    u = g * g
    p = g * (_S1 + u * (_S3 + u * (_S5 + u * (_S7 + u * _S9))))
    return jnp.where(f < 0.0, -p, p)


def _silu(x):
    return x * (0.5 * jnp.tanh(0.5 * x) + 0.5)


def _prep_body(h_ref, wa_ref, wb_ref, p_ref, q_ref):
    h = h_ref[...]
    p_ref[...] = jnp.dot(h, wa_ref[...], preferred_element_type=jnp.float32)
    q_ref[...] = jnp.dot(h, wb_ref[...], preferred_element_type=jnp.float32)


def _prep(h, wa, wb):
    n = h.shape[0]
    return pl.pallas_call(
        _prep_body,
        grid=(n // _BN,),
        in_specs=[
            pl.BlockSpec((_BN, _HID), lambda i: (i, 0)),
            pl.BlockSpec((_HID, _HID), lambda i: (0, 0)),
            pl.BlockSpec((_HID, _HID), lambda i: (0, 0)),
        ],
        out_specs=[
            pl.BlockSpec((_BN, _HID), lambda i: (i, 0)),
            pl.BlockSpec((_BN, _HID), lambda i: (i, 0)),
        ],
        out_shape=[
            jax.ShapeDtypeStruct((n, _HID), jnp.float32),
            jax.ShapeDtypeStruct((n, _HID), jnp.float32),
        ],
    )(h, wa, wb)


def _sc_gather_pq(p, q, src, dst):
    n, d = p.shape
    e = src.shape[0]
    per_w = e // _NW
    group = _UNROLL * _CHUNK
    ngroups = per_w // group
    mesh = plsc.VectorSubcoreMesh(core_axis_name="c", subcore_axis_name="s")

    scratch = ([pltpu.VMEM((group,), jnp.int32), pltpu.VMEM((group,), jnp.int32)]
               + [pltpu.VMEM((_CHUNK, d), jnp.float32) for _ in range(2 * _UNROLL)]
               + [pltpu.SemaphoreType.DMA, pltpu.SemaphoreType.DMA,
                  pltpu.SemaphoreType.DMA])

    @functools.partial(
        pl.kernel,
        mesh=mesh,
        out_type=[jax.ShapeDtypeStruct((e, d), jnp.float32),
                  jax.ShapeDtypeStruct((e, d), jnp.float32)],
        scratch_types=scratch,
    )
    def k(p_hbm, q_hbm, src_hbm, dst_hbm, g1_hbm, g2_hbm, *sc):
        si, di = sc[0], sc[1]
        r1 = sc[2:2 + _UNROLL]
        r2 = sc[2 + _UNROLL:2 + 2 * _UNROLL]
        isem, gsem, wsem = sc[-3], sc[-2], sc[-1]
        wid = lax.axis_index("s") * _NC + lax.axis_index("c")
        base = wid * per_w

        def body(t, carry):
            off0 = base + t * group
            ci = pltpu.async_copy(src_hbm.at[pl.ds(off0, group)], si, isem)
            cj = pltpu.async_copy(dst_hbm.at[pl.ds(off0, group)], di, isem)
            ci.wait()
            cj.wait()
            gs = []
            for u in range(_UNROLL):
                gs.append(pltpu.async_copy(
                    p_hbm.at[si.at[pl.ds(u * _CHUNK, _CHUNK)]], r1[u], gsem))
                gs.append(pltpu.async_copy(
                    q_hbm.at[di.at[pl.ds(u * _CHUNK, _CHUNK)]], r2[u], gsem))
            ws = []
            for u in range(_UNROLL):
                gs[2 * u].wait()
                gs[2 * u + 1].wait()
                off = off0 + u * _CHUNK
                ws.append(pltpu.async_copy(r1[u], g1_hbm.at[pl.ds(off, _CHUNK)], wsem))
                ws.append(pltpu.async_copy(r2[u], g2_hbm.at[pl.ds(off, _CHUNK)], wsem))
            for cpy in ws:
                cpy.wait()
            return carry

        lax.fori_loop(0, ngroups, body, 0)

    return k(p, q, src, dst)


def _sc_gather_pos(pos16, src, dst):
    n, d = pos16.shape
    e = src.shape[0]
    per_w = e // _NW
    group = _UNROLL * _CHUNK
    ngroups = per_w // group
    mesh = plsc.VectorSubcoreMesh(core_axis_name="c", subcore_axis_name="s")

    scratch = ([pltpu.VMEM((group,), jnp.int32), pltpu.VMEM((group,), jnp.int32)]
               + [pltpu.VMEM((_CHUNK, d), jnp.float32) for _ in range(2 * _UNROLL)]
               + [pltpu.SemaphoreType.DMA, pltpu.SemaphoreType.DMA,
                  pltpu.SemaphoreType.DMA])

    @functools.partial(
        pl.kernel,
        mesh=mesh,
        compiler_params=pltpu.CompilerParams(use_tc_tiling_on_sc=False),
        out_type=[jax.ShapeDtypeStruct((e, d), jnp.float32),
                  jax.ShapeDtypeStruct((e, d), jnp.float32)],
        scratch_types=scratch,
    )
    def k(t_hbm, src_hbm, dst_hbm, ps_hbm, pd_hbm, *sc):
        si, di = sc[0], sc[1]
        r1 = sc[2:2 + _UNROLL]
        r2 = sc[2 + _UNROLL:2 + 2 * _UNROLL]
        isem, gsem, wsem = sc[-3], sc[-2], sc[-1]
        wid = lax.axis_index("s") * _NC + lax.axis_index("c")
        base = wid * per_w

        def body(t, carry):
            off0 = base + t * group
            ci = pltpu.async_copy(src_hbm.at[pl.ds(off0, group)], si, isem)
            cj = pltpu.async_copy(dst_hbm.at[pl.ds(off0, group)], di, isem)
            ci.wait()
            cj.wait()
            gs = []
            for u in range(_UNROLL):
                gs.append(pltpu.async_copy(
                    t_hbm.at[si.at[pl.ds(u * _CHUNK, _CHUNK)]], r1[u], gsem))
                gs.append(pltpu.async_copy(
                    t_hbm.at[di.at[pl.ds(u * _CHUNK, _CHUNK)]], r2[u], gsem))
            ws = []
            for u in range(_UNROLL):
                gs[2 * u].wait()
                gs[2 * u + 1].wait()
                off = off0 + u * _CHUNK
                ws.append(pltpu.async_copy(r1[u], ps_hbm.at[pl.ds(off, _CHUNK)], wsem))
                ws.append(pltpu.async_copy(r2[u], pd_hbm.at[pl.ds(off, _CHUNK)], wsem))
            for cpy in ws:
                cpy.wait()
            return carry

        lax.fori_loop(0, ngroups, body, 0)

    return k(pos16, src, dst)


def _scatter_body(e, n, d, chunk, tiled):
    per_w = e // _NW
    group = _UNROLL * chunk
    ngroups = per_w // group
    rows_a, rows_b = 640, n - 15 * 640
    mesh = plsc.VectorSubcoreMesh(core_axis_name="c", subcore_axis_name="s")

    scratch = ([pltpu.VMEM((chunk,), jnp.int32) for _ in range(_UNROLL)]
               + [pltpu.VMEM((chunk, d), jnp.float32) for _ in range(_UNROLL)]
               + [pltpu.VMEM_SHARED((n, d), jnp.float32)]
               + [pltpu.SemaphoreType.DMA, pltpu.SemaphoreType.DMA,
                  pltpu.SemaphoreType.DMA])

    cp = None if tiled else pltpu.CompilerParams(use_tc_tiling_on_sc=False)

    @functools.partial(
        pl.kernel,
        mesh=mesh,
        compiler_params=cp,
        out_type=jax.ShapeDtypeStruct((_NC, n, d), jnp.float32),
        scratch_types=scratch,
    )
    def k(mm_hbm, dst_hbm, z_hbm, out_hbm, *sc):
        di = sc[0:_UNROLL]
        rv = sc[_UNROLL:2 * _UNROLL]
        acc = sc[2 * _UNROLL]
        isem, lsem, ssem = sc[-3], sc[-2], sc[-1]
        c = lax.axis_index("c")
        s = lax.axis_index("s")
        wid = s * _NC + c

        @pl.when(s < 15)
        def _():
            pltpu.sync_copy(z_hbm.at[pl.ds(s * rows_a, rows_a)],
                            acc.at[pl.ds(s * rows_a, rows_a)])

        @pl.when(s == 15)
        def _():
            pltpu.sync_copy(z_hbm.at[pl.ds(15 * rows_a, rows_b)],
                            acc.at[pl.ds(15 * rows_a, rows_b)])

        plsc.subcore_barrier()
        base = wid * per_w

        def body(t, carry):
            off0 = base + t * group
            ics, lcs = [], []
            for u in range(_UNROLL):
                off = off0 + u * chunk
                ics.append(pltpu.async_copy(dst_hbm.at[pl.ds(off, chunk)], di[u], isem))
                lcs.append(pltpu.async_copy(mm_hbm.at[pl.ds(off, chunk)], rv[u], lsem))
            scs = []
            for u in range(_UNROLL):
                ics[u].wait()
                lcs[u].wait()
                scs.append(pltpu.async_copy(rv[u], acc.at[di[u]], ssem, add=True))
            for cpy in scs:
                cpy.wait()
            return carry

        lax.fori_loop(0, ngroups, body, 0)
        plsc.subcore_barrier()

        @pl.when(s < 15)
        def _():
            pltpu.sync_copy(acc.at[pl.ds(s * rows_a, rows_a)],
                            out_hbm.at[c, pl.ds(s * rows_a, rows_a)])

        @pl.when(s == 15)
        def _():
            pltpu.sync_copy(acc.at[pl.ds(15 * rows_a, rows_b)],
                            out_hbm.at[c, pl.ds(15 * rows_a, rows_b)])

    return k


def _sc_scatter_m(m, dst, zeros128):
    e, d = m.shape
    n = zeros128.shape[0]
    return _scatter_body(e, n, d, 40, tiled=True)(m, dst, zeros128)


def _sc_scatter_msg(msg, dst, zeros16):
    e, d = msg.shape
    n = zeros16.shape[0]
    return _scatter_body(e, n, d, 80, tiled=False)(msg, dst, zeros16)


def _edge_body(g1_ref, g2_ref, ps_ref, pd_ref,
               w1c_ref, b1_ref, w2_ref, b2_ref,
               cw1_ref, cb1_ref, cw2t_ref,
               m_ref, msg_ref, loss_ref):
    ps = ps_ref[...]
    pd = pd_ref[...]
    rel = pd - ps
    d2 = jnp.sum(rel * rel, axis=1, keepdims=True) + 1e-8
    dist = jnp.sqrt(d2)
    freqs = (lax.broadcasted_iota(jnp.int32, (1, _NFREQ), 1) + 1
             ).astype(jnp.float32) * (math.pi / _CUTOFF)
    demb = _fast_sin(dist * freqs)
    pre = (g1_ref[...] + g2_ref[...]
           + jnp.dot(demb, w1c_ref[...], preferred_element_type=jnp.float32)
           + b1_ref[...])
    m1 = _silu(pre)
    m = _silu(jnp.dot(m1, w2_ref[...], preferred_element_type=jnp.float32)
              + b2_ref[...])
    w = 0.5 * (_fast_sin(dist * (math.pi / _CUTOFF) + (0.5 * math.pi)) + 1.0)
    w = w * (dist < _CUTOFF).astype(jnp.float32)
    m = m * w
    ch = _silu(jnp.dot(m, cw1_ref[...], preferred_element_type=jnp.float32)
               + cb1_ref[...])
    cs = jnp.tanh(jnp.sum(ch * cw2t_ref[...], axis=1, keepdims=True))
    m_ref[...] = m
    msg_ref[...] = rel / (dist + 1.0) * cs
    part = jnp.sum((dist - 1.5) ** 2 * w)

    @pl.when(pl.program_id(0) == 0)
    def _():
        loss_ref[...] = jnp.zeros_like(loss_ref)

    loss_ref[...] += part


def _edge(g1, g2, ps, pd, params, l):
    e = g1.shape[0]
    w1c = params[f"e{l}_W1"][2 * _HID:]
    b1 = params[f"e{l}_b1"].reshape(1, _HID)
    w2 = params[f"e{l}_W2"]
    b2 = params[f"e{l}_b2"].reshape(1, _HID)
    cw1 = params[f"c{l}_W1"]
    cb1 = params[f"c{l}_b1"].reshape(1, _HID)
    cw2t = params[f"c{l}_W2"].reshape(1, _HID)
    full = lambda shape: pl.BlockSpec(shape, lambda i: (0,) * len(shape))
    return pl.pallas_call(
        _edge_body,
        grid=(e // _BE,),
        in_specs=[
            pl.BlockSpec((_BE, _HID), lambda i: (i, 0)),
            pl.BlockSpec((_BE, _HID), lambda i: (i, 0)),
            pl.BlockSpec((_BE, 16), lambda i: (i, 0)),
            pl.BlockSpec((_BE, 16), lambda i: (i, 0)),
            full((_NFREQ, _HID)), full((1, _HID)), full((_HID, _HID)),
            full((1, _HID)), full((_HID, _HID)), full((1, _HID)),
            full((1, _HID)),
        ],
        out_specs=[
            pl.BlockSpec((_BE, _HID), lambda i: (i, 0)),
            pl.BlockSpec((_BE, 16), lambda i: (i, 0)),
            pl.BlockSpec((1, 1), lambda i: (0, 0)),
        ],
        out_shape=[
            jax.ShapeDtypeStruct((e, _HID), jnp.float32),
            jax.ShapeDtypeStruct((e, 16), jnp.float32),
            jax.ShapeDtypeStruct((1, 1), jnp.float32),
        ],
    )(g1, g2, ps, pd, w1c, b1, w2, b2, cw1, cb1, cw2t)


def _node_body(h_ref, a0_ref, a1_ref, p16_ref, d0_ref, d1_ref,
               nw1a_ref, nw1b_ref, nb1_ref, nw2_ref, nb2_ref,
               wa_ref, wb_ref,
               hn_ref, pn_ref, pp_ref, pq_ref):
    h = h_ref[...]
    agg = a0_ref[0] + a1_ref[0]
    upd = _silu(jnp.dot(h, nw1a_ref[...], preferred_element_type=jnp.float32)
                + jnp.dot(agg, nw1b_ref[...], preferred_element_type=jnp.float32)
                + nb1_ref[...])
    hn = h + jnp.dot(upd, nw2_ref[...], preferred_element_type=jnp.float32) + nb2_ref[...]
    hn_ref[...] = hn
    pn_ref[...] = p16_ref[...] + d0_ref[0] + d1_ref[0]
    pp_ref[...] = jnp.dot(hn, wa_ref[...], preferred_element_type=jnp.float32)
    pq_ref[...] = jnp.dot(hn, wb_ref[...], preferred_element_type=jnp.float32)


def _node(h, agg2, pos16, dpos2, params, l, wa, wb):
    n = h.shape[0]
    nw1 = params[f"n{l}_W1"]
    nb1 = params[f"n{l}_b1"].reshape(1, _HID)
    nw2 = params[f"n{l}_W2"]
    nb2 = params[f"n{l}_b2"].reshape(1, _HID)
    full = lambda shape: pl.BlockSpec(shape, lambda i: (0,) * len(shape))
    return pl.pallas_call(
        _node_body,
        grid=(n // _BN,),
        in_specs=[
            pl.BlockSpec((_BN, _HID), lambda i: (i, 0)),
            pl.BlockSpec((1, _BN, _HID), lambda i: (0, i, 0)),
            pl.BlockSpec((1, _BN, _HID), lambda i: (1, i, 0)),
            pl.BlockSpec((_BN, 16), lambda i: (i, 0)),
            pl.BlockSpec((1, _BN, 16), lambda i: (0, i, 0)),
            pl.BlockSpec((1, _BN, 16), lambda i: (1, i, 0)),
            full((_HID, _HID)), full((_HID, _HID)), full((1, _HID)),
            full((_HID, _HID)), full((1, _HID)),
            full((_HID, _HID)), full((_HID, _HID)),
        ],
        out_specs=[
            pl.BlockSpec((_BN, _HID), lambda i: (i, 0)),
            pl.BlockSpec((_BN, 16), lambda i: (i, 0)),
            pl.BlockSpec((_BN, _HID), lambda i: (i, 0)),
            pl.BlockSpec((_BN, _HID), lambda i: (i, 0)),
        ],
        out_shape=[
            jax.ShapeDtypeStruct((n, _HID), jnp.float32),
            jax.ShapeDtypeStruct((n, 16), jnp.float32),
            jax.ShapeDtypeStruct((n, _HID), jnp.float32),
            jax.ShapeDtypeStruct((n, _HID), jnp.float32),
        ],
    )(h, agg2, agg2, pos16, dpos2, dpos2,
      nw1[:_HID], nw1[_HID:], nb1, nw2, nb2, wa, wb)


def _psum_body(p16_ref, s_ref):
    @pl.when(pl.program_id(0) == 0)
    def _():
        s_ref[...] = jnp.zeros_like(s_ref)

    s_ref[...] += jnp.sum(p16_ref[...], axis=0, keepdims=True)


def _psum(pos16):
    n = pos16.shape[0]
    return pl.pallas_call(
        _psum_body,
        grid=(n // _BN,),
        in_specs=[pl.BlockSpec((_BN, 16), lambda i: (i, 0))],
        out_specs=pl.BlockSpec((1, 16), lambda i: (0, 0)),
        out_shape=jax.ShapeDtypeStruct((1, 16), jnp.float32),
    )(pos16)


def _head_body(gp_ref, ip_ref, p16_ref, psum_ref,
               gb1_ref, gw2_ref, gb2_ref,
               iwn_ref, ib1_ref, iw2_ref, ib2_ref,
               geo_ref, inv_ref, inv_n):
    s = _silu(gp_ref[...] + gb1_ref[...])
    geo_ref[...] = jnp.dot(s, gw2_ref[...], preferred_element_type=jnp.float32) + gb2_ref[...]
    p16 = p16_ref[...]
    norm = jnp.sqrt(jnp.sum(p16 * p16, axis=1, keepdims=True))
    mean = psum_ref[...] * inv_n
    pre = (ip_ref[...] + ib1_ref[...]
           + norm * iwn_ref[0:1, :]
           + mean[0:1, 0:1] * iwn_ref[1:2, :]
           + mean[0:1, 1:2] * iwn_ref[2:3, :]
           + mean[0:1, 2:3] * iwn_ref[3:4, :])
    inv_ref[...] = jnp.dot(_silu(pre), iw2_ref[...], preferred_element_type=jnp.float32) + ib2_ref[...]


def _head(gp, ip, pos16, psum, params):
    n = gp.shape[0]
    gb1 = params["g_b1"].reshape(1, _HID)
    gw2 = params["g_W2"]
    gb2 = params["g_b2"].reshape(1, 64)
    iwn = jnp.concatenate([params["i_W1"][_HID:], jnp.zeros((4, _HID), jnp.float32)], axis=0)
    ib1 = params["i_b1"].reshape(1, _HID)
    iw2 = params["i_W2"]
    ib2 = params["i_b2"].reshape(1, _HID)
    full = lambda shape: pl.BlockSpec(shape, lambda i: (0,) * len(shape))
    return pl.pallas_call(
        functools.partial(_head_body, inv_n=1.0 / n),
        grid=(n // _BN,),
        in_specs=[
            pl.BlockSpec((_BN, _HID), lambda i: (i, 0)),
            pl.BlockSpec((_BN, _HID), lambda i: (i, 0)),
            pl.BlockSpec((_BN, 16), lambda i: (i, 0)),
            full((1, 16)),
            full((1, _HID)), full((_HID, 64)), full((1, 64)),
            full((8, _HID)), full((1, _HID)), full((_HID, _HID)), full((1, _HID)),
        ],
        out_specs=[
            pl.BlockSpec((_BN, 64), lambda i: (i, 0)),
            pl.BlockSpec((_BN, _HID), lambda i: (i, 0)),
        ],
        out_shape=[
            jax.ShapeDtypeStruct((n, 64), jnp.float32),
            jax.ShapeDtypeStruct((n, _HID), jnp.float32),
        ],
    )(gp, ip, pos16, psum, gb1, gw2, gb2, iwn, ib1, iw2, ib2)


def kernel(h, pos, batch, edge_index, params):
    del batch
    n = h.shape[0]
    e = edge_index.shape[1]
    src = edge_index[0].astype(jnp.int32)
    dst = edge_index[1].astype(jnp.int32)
    pos16 = jnp.concatenate([pos.astype(jnp.float32),
                             jnp.zeros((n, 13), jnp.float32)], axis=1)
    h = h.astype(jnp.float32)
    zeros128 = jnp.zeros((n, _HID), jnp.float32)
    zeros16 = jnp.zeros((n, 16), jnp.float32)

    p, q = _prep(h, params["e0_W1"][:_HID], params["e0_W1"][_HID:2 * _HID])
    losses = []
    for l in range(_NL):
        g1, g2 = _sc_gather_pq(p, q, src, dst)
        ps, pd = _sc_gather_pos(pos16, src, dst)
        m, msg, lpart = _edge(g1, g2, ps, pd, params, l)
        agg2 = _sc_scatter_m(m, dst, zeros128)
        dpos2 = _sc_scatter_msg(msg, dst, zeros16)
        if l < _NL - 1:
            wa = params[f"e{l + 1}_W1"][:_HID]
            wb = params[f"e{l + 1}_W1"][_HID:2 * _HID]
        else:
            wa = params["g_W1"]
            wb = params["i_W1"][:_HID]
        h, pos16, p, q = _node(h, agg2, pos16, dpos2, params, l, wa, wb)
        losses.append(lpart)

    psum = _psum(pos16)
    geo, inv = _head(p, q, pos16, psum, params)
    closs = ((losses[0] + losses[1] + losses[2] + losses[3])[0, 0] / e).astype(jnp.float32)
    return h, pos16[:, :3], geo, inv, closs
